# baseline (speedup 1.0000x reference)
import functools

import jax
import jax.numpy as jnp
from jax import lax
from jax.experimental import pallas as pl
from jax.experimental.pallas import tpu as pltpu
from jax.experimental.pallas import tpu_sc as plsc

N_NODES = 10000
N_EDGES = 320000
N_WORKERS = 32
E_PER_W = N_EDGES // N_WORKERS
CHUNK = 80
N_CHUNKS = E_PER_W // CHUNK
ROWS_PER_TILE = N_NODES // 16

_MESH = dict(core_axis_name="c", subcore_axis_name="s")
_SC_PARAMS = pltpu.CompilerParams(needs_layout_passes=False)


_RSQRT_MAGIC = 0x5F3759DF


def _newton_rsqrt(d):
    i = plsc.bitcast(d, jnp.int32)
    y = plsc.bitcast(_RSQRT_MAGIC - lax.shift_right_logical(i, 1), jnp.float32)
    for _ in range(3):
        y = y * (1.5 - 0.5 * d * y * y)
    return y


def _sc_pre_body(x_hbm, dst_hbm, dinv_hbm, g_hbm, dstv, hist, degbuf, dinvb,
                 xbuf, gbuf, iota80, degsp):
    c = lax.axis_index("c")
    s = lax.axis_index("s")

    @pl.when(c == 0)
    def _():
        pltpu.sync_copy(dst_hbm.at[s], dstv)

        def zero(i, _):
            for k in range(8):
                hist[i, pl.ds(k * 16, 16)] = jnp.zeros((16,), jnp.int32)
            return 0

        lax.fori_loop(0, 80, zero, 0)
        pltpu.sync_copy(hist.at[pl.ds(0, 5)], degsp.at[pl.ds(5 * s, 5)])

    plsc.subcore_barrier()

    @pl.when(c == 0)
    def _():
        ones = jnp.ones((16,), jnp.int32)

        def step(i, _):
            idx = dstv[pl.ds(i * 16, 16)]
            row = lax.shift_right_logical(idx, 7)
            col = jnp.bitwise_and(idx, 127)
            plsc.addupdate_scatter(hist, [row, col], ones)
            return 0

        lax.fori_loop(0, (2 * E_PER_W) // 16, step, 0)
        for k in range(5):
            iota80[pl.ds(k * 16, 16)] = k * 16 + lax.iota(jnp.int32, 16)
        pltpu.sync_copy(hist, degsp.at[iota80], add=True)

    plsc.subcore_barrier()

    @pl.when(c == 0)
    def _():
        pltpu.sync_copy(degsp.at[pl.ds(5 * s, 5)], degbuf)
        for r in range(5):
            for k in range(8):
                d = degbuf[r, pl.ds(k * 16, 16)].astype(jnp.float32) + 1.0
                dinvb[pl.ds(r * 128 + k * 16, 16)] = _newton_rsqrt(d)
        pltpu.sync_copy(dinvb, dinv_hbm.at[s])

        node0 = s * 640
        zeros16 = jnp.zeros((16,), jnp.int32)

        def do_chunk(t):
            pltpu.sync_copy(x_hbm.at[pl.ds(node0 + t * 80, 80)], xbuf)

            def srow(r, _):
                dval = plsc.load_gather(dinvb, [zeros16 + (t * 80 + r)])
                for k in range(8):
                    gbuf[r, pl.ds(k * 16, 16)] = (
                        xbuf[r, pl.ds(k * 16, 16)] * dval)
                return 0

            lax.fori_loop(0, 80, srow, 0)
            pltpu.sync_copy(gbuf, g_hbm.at[pl.ds(node0 + t * 80, 80)])

        for t in range(8):
            if t < 5:
                do_chunk(t)
            else:
                @pl.when(s < 15)
                def _():
                    do_chunk(t)


def _sc_pre(x, dst16):
    k = pl.kernel(
        _sc_pre_body,
        out_type=(
            jax.ShapeDtypeStruct((16, 640), jnp.float32),
            jax.ShapeDtypeStruct((N_NODES, 128), jnp.float32),
        ),
        scratch_types=[
            pltpu.VMEM((2 * E_PER_W,), jnp.int32),
            pltpu.VMEM((80, 128), jnp.int32),
            pltpu.VMEM((5, 128), jnp.int32),
            pltpu.VMEM((640,), jnp.float32),
            pltpu.VMEM((80, 128), jnp.float32),
            pltpu.VMEM((80, 128), jnp.float32),
            pltpu.VMEM((80,), jnp.int32),
            pltpu.VMEM_SHARED((80, 128), jnp.int32),
        ],
        mesh=plsc.VectorSubcoreMesh(**_MESH),
        compiler_params=_SC_PARAMS,
    )
    return k(x, dst16)


def _sc_agg_body(g_hbm, src_hbm, dst_hbm, out_hbm, srcv, dstv,
                 buf0, buf1, acc, sem0, sem1):
    c = lax.axis_index("c")
    s = lax.axis_index("s")
    wid = s * 2 + c
    pltpu.sync_copy(src_hbm.at[wid], srcv)
    pltpu.sync_copy(dst_hbm.at[wid], dstv)
    base = s * ROWS_PER_TILE

    @pl.when(c == 0)
    def _():
        pltpu.sync_copy(g_hbm.at[pl.ds(s * 624, 624)],
                        acc.at[pl.ds(s * 624, 624)])

    @pl.when(jnp.logical_and(c == 0, s == 15))
    def _():
        pltpu.sync_copy(g_hbm.at[pl.ds(9984, 16)], acc.at[pl.ds(9984, 16)])

    @pl.when(c != 0)
    def _():
        def zbuf(r, _):
            for k in range(8):
                buf0[r, pl.ds(k * 16, 16)] = jnp.zeros((16,), jnp.float32)
            return 0

        lax.fori_loop(0, CHUNK, zbuf, 0)
        for t in range(7):
            pltpu.sync_copy(buf0, acc.at[pl.ds(base + t * CHUNK, CHUNK)])
        pltpu.sync_copy(buf0.at[pl.ds(0, 65)], acc.at[pl.ds(base + 560, 65)])

    plsc.subcore_barrier()

    def g_at(j):
        return g_hbm.at[srcv.at[pl.ds(j * CHUNK, CHUNK)]]

    pltpu.async_copy(g_at(0), buf0, sem0)

    def pair(i, _):
        j = 2 * i
        pltpu.async_copy(g_at(j + 1), buf1, sem1)
        pltpu.make_async_copy(g_at(j), buf0, sem0).wait()
        pltpu.sync_copy(buf0, acc.at[dstv.at[j]], add=True)
        pltpu.async_copy(g_at(j + 2), buf0, sem0)
        pltpu.make_async_copy(g_at(j + 1), buf1, sem1).wait()
        pltpu.sync_copy(buf1, acc.at[dstv.at[j + 1]], add=True)
        return 0

    lax.fori_loop(0, (N_CHUNKS - 1) // 2, pair, 0)
    j = N_CHUNKS - 1
    pltpu.make_async_copy(g_at(j), buf0, sem0).wait()
    pltpu.sync_copy(buf0, acc.at[dstv.at[j]], add=True)
    plsc.subcore_barrier()
    pltpu.sync_copy(acc.at[pl.ds(base, ROWS_PER_TILE)], out_hbm.at[c, s])


def _sc_agg(g, src32, dst32):
    k = pl.kernel(
        _sc_agg_body,
        out_type=jax.ShapeDtypeStruct((2, 16, ROWS_PER_TILE, 128), jnp.float32),
        scratch_types=[
            pltpu.VMEM((E_PER_W,), jnp.int32),
            pltpu.VMEM((N_CHUNKS, CHUNK), jnp.int32),
            pltpu.VMEM((CHUNK, 128), jnp.float32),
            pltpu.VMEM((CHUNK, 128), jnp.float32),
            pltpu.VMEM_SHARED((N_NODES, 128), jnp.float32),
            pltpu.SemaphoreType.DMA,
            pltpu.SemaphoreType.DMA,
        ],
        mesh=plsc.VectorSubcoreMesh(**_MESH),
        compiler_params=_SC_PARAMS,
    )
    return k(g, src32, dst32)


_BLK = 1000


def _tc_mid_body(p0, p1, d, w1, bb1, w2, o_ref):
    a = d[...] * (p0[...] + p1[...])
    h = jnp.dot(a, w1[...], preferred_element_type=jnp.float32) + bb1[...]
    h = jnp.maximum(h, 0.0)
    t = jnp.dot(h, w2[...], preferred_element_type=jnp.float32)
    o_ref[...] = d[...] * t


def _tc_mid(p0, p1, dinv_col, W1, b1, W2):
    grid = (N_NODES // _BLK,)
    row = lambda i: (i, 0)
    full = lambda i: (0, 0)
    return pl.pallas_call(
        _tc_mid_body,
        grid=grid,
        in_specs=[
            pl.BlockSpec((_BLK, 128), row),
            pl.BlockSpec((_BLK, 128), row),
            pl.BlockSpec((_BLK, 1), row),
            pl.BlockSpec((128, 256), full),
            pl.BlockSpec((1, 256), full),
            pl.BlockSpec((256, 128), full),
        ],
        out_specs=pl.BlockSpec((_BLK, 128), row),
        out_shape=jax.ShapeDtypeStruct((N_NODES, 128), jnp.float32),
    )(p0, p1, dinv_col, W1, b1.reshape(1, 256), W2)


def _tc_post_body(q0, q1, d, bb2, o_ref):
    a = d[...] * (q0[...] + q1[...]) + bb2[...]
    m = jnp.max(a, axis=-1, keepdims=True)
    e = jnp.exp(a - m)
    o_ref[...] = e / jnp.sum(e, axis=-1, keepdims=True)


def _tc_post(q0, q1, dinv_col, b2):
    grid = (N_NODES // _BLK,)
    row = lambda i: (i, 0)
    full = lambda i: (0, 0)
    return pl.pallas_call(
        _tc_post_body,
        grid=grid,
        in_specs=[
            pl.BlockSpec((_BLK, 128), row),
            pl.BlockSpec((_BLK, 128), row),
            pl.BlockSpec((_BLK, 1), row),
            pl.BlockSpec((1, 128), full),
        ],
        out_specs=pl.BlockSpec((_BLK, 128), row),
        out_shape=jax.ShapeDtypeStruct((N_NODES, 128), jnp.float32),
    )(q0, q1, dinv_col, b2.reshape(1, 128))


def kernel(x, edge_index, W1, b1, W2, b2):
    src = edge_index[0].astype(jnp.int32)
    dst = edge_index[1].astype(jnp.int32)
    src_r = src.reshape(N_WORKERS, E_PER_W)
    dst_r = dst.reshape(N_WORKERS, N_CHUNKS, CHUNK)
    dst16 = dst.reshape(16, 2 * E_PER_W)

    dinv16, g1 = _sc_pre(x, dst16)
    dinv_col = dinv16.reshape(16 * 640)[:N_NODES].reshape(N_NODES, 1)

    p = _sc_agg(g1, src_r, dst_r).reshape(2, N_NODES, 128)
    g2 = _tc_mid(p[0], p[1], dinv_col, W1, b1, W2)
    q = _sc_agg(g2, src_r, dst_r).reshape(2, N_NODES, 128)
    out = _tc_post(q[0], q[1], dinv_col, b2)
    return out


# speedup vs baseline: 1.0817x; 1.0817x over previous
import jax
import jax.numpy as jnp
from jax import lax
from jax.experimental import pallas as pl
from jax.experimental.pallas import tpu as pltpu
from jax.experimental.pallas import tpu_sc as plsc

N_NODES = 10000
N_EDGES = 320000
N_WORKERS = 32
E_PER_W = N_EDGES // N_WORKERS
CHUNK = 80
N_CHUNKS = E_PER_W // CHUNK
ROWS_PER_TILE = N_NODES // 16

_MESH = dict(core_axis_name="c", subcore_axis_name="s")
_SC_PARAMS = pltpu.CompilerParams(needs_layout_passes=False)


def _sc_deg_body(dst_hbm, out_hbm, dstv, hist):
    c = lax.axis_index("c")
    s = lax.axis_index("s")
    wid = s * 2 + c
    pltpu.sync_copy(dst_hbm.at[wid], dstv)

    def zero(i, _):
        hist[pl.ds(i * 16, 16)] = jnp.zeros((16,), jnp.int32)
        return 0

    lax.fori_loop(0, N_NODES // 16, zero, 0)

    ones = jnp.ones((16,), jnp.int32)

    def step(i, _):
        idx = dstv[pl.ds(i * 16, 16)]
        plsc.addupdate_scatter(hist, [idx], ones)
        return 0

    lax.fori_loop(0, E_PER_W // 16, step, 0)
    pltpu.sync_copy(hist, out_hbm.at[wid])


def _sc_deg(dst32):
    k = pl.kernel(
        _sc_deg_body,
        out_type=jax.ShapeDtypeStruct((N_WORKERS, N_NODES), jnp.int32),
        scratch_types=[
            pltpu.VMEM((E_PER_W,), jnp.int32),
            pltpu.VMEM((N_NODES,), jnp.int32),
        ],
        mesh=plsc.VectorSubcoreMesh(**_MESH),
        compiler_params=_SC_PARAMS,
    )
    return k(dst32)


def _sc_agg_body(g_hbm, src_hbm, dst_hbm, out_hbm, srcv, dst0, dst1,
                 buf0, buf1, acc, sem0, sem1, semd0, semd1):
    c = lax.axis_index("c")
    s = lax.axis_index("s")
    wid = s * 2 + c
    pltpu.sync_copy(src_hbm.at[wid], srcv)

    def zbuf(r, _):
        for k in range(8):
            buf0[r, pl.ds(k * 16, 16)] = jnp.zeros((16,), jnp.float32)
        return 0

    lax.fori_loop(0, CHUNK, zbuf, 0)
    base = s * ROWS_PER_TILE
    for t in range(7):
        pltpu.sync_copy(buf0, acc.at[pl.ds(base + t * CHUNK, CHUNK)])
    pltpu.sync_copy(buf0.at[pl.ds(0, 65)], acc.at[pl.ds(base + 560, 65)])
    plsc.subcore_barrier()

    def g_at(j):
        return g_hbm.at[srcv.at[pl.ds(j * CHUNK, CHUNK)]]

    pltpu.async_copy(g_at(0), buf0, sem0)
    pltpu.async_copy(dst_hbm.at[wid * N_CHUNKS], dst0, semd0)

    def pair(i, _):
        j = 2 * i
        pltpu.async_copy(g_at(j + 1), buf1, sem1)
        pltpu.async_copy(dst_hbm.at[wid * N_CHUNKS + j + 1], dst1, semd1)
        pltpu.make_async_copy(g_at(j), buf0, sem0).wait()
        pltpu.make_async_copy(dst_hbm.at[wid * N_CHUNKS + j], dst0, semd0).wait()
        pltpu.sync_copy(buf0, acc.at[dst0], add=True)
        pltpu.async_copy(g_at(j + 2), buf0, sem0)
        pltpu.async_copy(dst_hbm.at[wid * N_CHUNKS + j + 2], dst0, semd0)
        pltpu.make_async_copy(g_at(j + 1), buf1, sem1).wait()
        pltpu.make_async_copy(dst_hbm.at[wid * N_CHUNKS + j + 1], dst1, semd1).wait()
        pltpu.sync_copy(buf1, acc.at[dst1], add=True)
        return 0

    lax.fori_loop(0, (N_CHUNKS - 1) // 2, pair, 0)
    j = N_CHUNKS - 1
    pltpu.make_async_copy(g_at(j), buf0, sem0).wait()
    pltpu.make_async_copy(dst_hbm.at[wid * N_CHUNKS + j], dst0, semd0).wait()
    pltpu.sync_copy(buf0, acc.at[dst0], add=True)
    plsc.subcore_barrier()
    pltpu.sync_copy(acc.at[pl.ds(base, ROWS_PER_TILE)], out_hbm.at[c, s])


def _sc_agg(g, src32, dst32):
    k = pl.kernel(
        _sc_agg_body,
        out_type=jax.ShapeDtypeStruct((2, 16, ROWS_PER_TILE, 128), jnp.float32),
        scratch_types=[
            pltpu.VMEM((E_PER_W,), jnp.int32),
            pltpu.VMEM((CHUNK,), jnp.int32),
            pltpu.VMEM((CHUNK,), jnp.int32),
            pltpu.VMEM((CHUNK, 128), jnp.float32),
            pltpu.VMEM((CHUNK, 128), jnp.float32),
            pltpu.VMEM_SHARED((N_NODES, 128), jnp.float32),
            pltpu.SemaphoreType.DMA,
            pltpu.SemaphoreType.DMA,
            pltpu.SemaphoreType.DMA,
            pltpu.SemaphoreType.DMA,
        ],
        mesh=plsc.VectorSubcoreMesh(**_MESH),
        compiler_params=_SC_PARAMS,
    )
    return k(g, src32, dst32)


def _tc_dinv_body(h_ref, o_ref):
    deg = jnp.sum(h_ref[...], axis=0).astype(jnp.float32) + 1.0
    o_ref[...] = lax.rsqrt(deg)


def _tc_dinv(hists):
    return pl.pallas_call(
        _tc_dinv_body,
        out_shape=jax.ShapeDtypeStruct((N_NODES,), jnp.float32),
    )(hists)


_BLK = 1000


def _tc_scale_body(d_ref, x_ref, o_ref):
    o_ref[...] = d_ref[...] * x_ref[...]


def _tc_scale(dinv_col, x):
    grid = (N_NODES // _BLK,)
    return pl.pallas_call(
        _tc_scale_body,
        grid=grid,
        in_specs=[
            pl.BlockSpec((_BLK, 1), lambda i: (i, 0)),
            pl.BlockSpec((_BLK, 128), lambda i: (i, 0)),
        ],
        out_specs=pl.BlockSpec((_BLK, 128), lambda i: (i, 0)),
        out_shape=jax.ShapeDtypeStruct((N_NODES, 128), jnp.float32),
    )(dinv_col, x)


def _tc_mid_body(p0, p1, g1, d, w1, bb1, w2, o_ref):
    a = d[...] * (p0[...] + p1[...] + g1[...])
    h = jnp.dot(a, w1[...], preferred_element_type=jnp.float32) + bb1[...]
    h = jnp.maximum(h, 0.0)
    t = jnp.dot(h, w2[...], preferred_element_type=jnp.float32)
    o_ref[...] = d[...] * t


def _tc_mid(p0, p1, g1, dinv_col, W1, b1, W2):
    grid = (N_NODES // _BLK,)
    row = lambda i: (i, 0)
    full = lambda i: (0, 0)
    return pl.pallas_call(
        _tc_mid_body,
        grid=grid,
        in_specs=[
            pl.BlockSpec((_BLK, 128), row),
            pl.BlockSpec((_BLK, 128), row),
            pl.BlockSpec((_BLK, 128), row),
            pl.BlockSpec((_BLK, 1), row),
            pl.BlockSpec((128, 256), full),
            pl.BlockSpec((1, 256), full),
            pl.BlockSpec((256, 128), full),
        ],
        out_specs=pl.BlockSpec((_BLK, 128), row),
        out_shape=jax.ShapeDtypeStruct((N_NODES, 128), jnp.float32),
    )(p0, p1, g1, dinv_col, W1, b1.reshape(1, 256), W2)


def _tc_post_body(q0, q1, g2, d, bb2, o_ref):
    a = d[...] * (q0[...] + q1[...] + g2[...]) + bb2[...]
    m = jnp.max(a, axis=-1, keepdims=True)
    e = jnp.exp(a - m)
    o_ref[...] = e / jnp.sum(e, axis=-1, keepdims=True)


def _tc_post(q0, q1, g2, dinv_col, b2):
    grid = (N_NODES // _BLK,)
    row = lambda i: (i, 0)
    full = lambda i: (0, 0)
    return pl.pallas_call(
        _tc_post_body,
        grid=grid,
        in_specs=[
            pl.BlockSpec((_BLK, 128), row),
            pl.BlockSpec((_BLK, 128), row),
            pl.BlockSpec((_BLK, 128), row),
            pl.BlockSpec((_BLK, 1), row),
            pl.BlockSpec((1, 128), full),
        ],
        out_specs=pl.BlockSpec((_BLK, 128), row),
        out_shape=jax.ShapeDtypeStruct((N_NODES, 128), jnp.float32),
    )(q0, q1, g2, dinv_col, b2.reshape(1, 128))


def kernel(x, edge_index, W1, b1, W2, b2):
    src = edge_index[0].astype(jnp.int32)
    dst = edge_index[1].astype(jnp.int32)
    src_r = src.reshape(N_WORKERS, E_PER_W)
    dst_r = dst.reshape(N_WORKERS * N_CHUNKS, CHUNK)
    dst_flat = dst.reshape(N_WORKERS, E_PER_W)

    hists = _sc_deg(dst_flat)
    dinv = _tc_dinv(hists)
    dinv_col = dinv.reshape(N_NODES, 1)

    g1 = _tc_scale(dinv_col, x)
    p = _sc_agg(g1, src_r, dst_r).reshape(2, N_NODES, 128)
    g2 = _tc_mid(p[0], p[1], g1, dinv_col, W1, b1, W2)
    q = _sc_agg(g2, src_r, dst_r).reshape(2, N_NODES, 128)
    out = _tc_post(q[0], q[1], g2, dinv_col, b2)
    return out


# speedup vs baseline: 1.2093x; 1.1180x over previous
import jax
import jax.numpy as jnp
from jax import lax
from jax.experimental import pallas as pl
from jax.experimental.pallas import tpu as pltpu
from jax.experimental.pallas import tpu_sc as plsc

N_NODES = 10000
N_EDGES = 320000
N_WORKERS = 32
E_PER_W = N_EDGES // N_WORKERS
CHUNK = 80
N_CHUNKS = E_PER_W // CHUNK
ROWS_PER_TILE = N_NODES // 16

_MESH = dict(core_axis_name="c", subcore_axis_name="s")
_SC_PARAMS = pltpu.CompilerParams(needs_layout_passes=False)


def _sc_deg_body(dst_hbm, out_hbm, dstv, hist):
    c = lax.axis_index("c")
    s = lax.axis_index("s")
    wid = s * 2 + c
    pltpu.sync_copy(dst_hbm.at[wid], dstv)

    def zero(i, _):
        hist[pl.ds(i * 16, 16)] = jnp.zeros((16,), jnp.int32)
        return 0

    lax.fori_loop(0, N_NODES // 16, zero, 0)

    ones = jnp.ones((16,), jnp.int32)

    def step(i, _):
        idx = dstv[pl.ds(i * 16, 16)]
        plsc.addupdate_scatter(hist, [idx], ones)
        return 0

    lax.fori_loop(0, E_PER_W // 16, step, 0)
    pltpu.sync_copy(hist, out_hbm.at[wid])


def _sc_deg(dst32):
    k = pl.kernel(
        _sc_deg_body,
        out_type=jax.ShapeDtypeStruct((N_WORKERS, N_NODES), jnp.int32),
        scratch_types=[
            pltpu.VMEM((E_PER_W,), jnp.int32),
            pltpu.VMEM((N_NODES,), jnp.int32),
        ],
        mesh=plsc.VectorSubcoreMesh(**_MESH),
        compiler_params=_SC_PARAMS,
    )
    return k(dst32)


def _sc_agg_body(g_hbm, src_hbm, dst_hbm, out_hbm, srcv,
                 d0, d1, d2, b0, b1, b2, acc,
                 sg0, sg1, sg2, sd0, sd1, sd2, ss0, ss1, ss2):
    c = lax.axis_index("c")
    s = lax.axis_index("s")
    wid = s * 2 + c
    pltpu.sync_copy(src_hbm.at[wid], srcv)

    def zbuf(r, _):
        for k in range(8):
            b0[r, pl.ds(k * 16, 16)] = jnp.zeros((16,), jnp.float32)
        return 0

    lax.fori_loop(0, CHUNK, zbuf, 0)
    base = s * ROWS_PER_TILE
    for t in range(7):
        pltpu.sync_copy(b0, acc.at[pl.ds(base + t * CHUNK, CHUNK)])
    pltpu.sync_copy(b0.at[pl.ds(0, 65)], acc.at[pl.ds(base + 560, 65)])
    plsc.subcore_barrier()

    D = (d0, d1, d2)
    B = (b0, b1, b2)
    SG = (sg0, sg1, sg2)
    SD = (sd0, sd1, sd2)
    SS = (ss0, ss1, ss2)

    def g_at(j):
        return g_hbm.at[srcv.at[pl.ds(j * CHUNK, CHUNK)]]

    def dst_at(j):
        return dst_hbm.at[wid * N_CHUNKS + j]

    def issue_gd(m, k):
        pltpu.async_copy(dst_at(m), D[k], SD[k])
        pltpu.async_copy(g_at(m), B[k], SG[k])

    def wait_gd(m, k):
        pltpu.make_async_copy(g_at(m), B[k], SG[k]).wait()
        pltpu.make_async_copy(dst_at(m), D[k], SD[k]).wait()

    def issue_s(k):
        pltpu.async_copy(B[k], acc.at[D[k]], SS[k], add=True)

    def wait_s(k):
        pltpu.make_async_copy(B[k], acc.at[D[k]], SS[k]).wait()

    issue_gd(0, 0)
    issue_gd(1, 1)
    wait_gd(0, 0)
    issue_s(0)
    issue_gd(2, 2)
    wait_gd(1, 1)
    issue_s(1)
    wait_s(0)
    issue_gd(3, 0)
    wait_gd(2, 2)
    issue_s(2)
    wait_s(1)
    issue_gd(4, 1)
    wait_gd(3, 0)
    issue_s(0)

    def tri(i, _):
        m = 4 + 3 * i
        wait_s(2)
        issue_gd(m + 1, 2)
        wait_gd(m, 1)
        issue_s(1)
        wait_s(0)
        issue_gd(m + 2, 0)
        wait_gd(m + 1, 2)
        issue_s(2)
        wait_s(1)
        issue_gd(m + 3, 1)
        wait_gd(m + 2, 0)
        issue_s(0)
        return 0

    lax.fori_loop(0, 40, tri, 0)
    wait_s(2)
    wait_gd(124, 1)
    issue_s(1)
    wait_s(0)
    wait_s(1)
    plsc.subcore_barrier()
    pltpu.sync_copy(acc.at[pl.ds(base, ROWS_PER_TILE)], out_hbm.at[c, s])


def _sc_agg(g, src32, dst32):
    k = pl.kernel(
        _sc_agg_body,
        out_type=jax.ShapeDtypeStruct((2, 16, ROWS_PER_TILE, 128), jnp.float32),
        scratch_types=[
            pltpu.VMEM((E_PER_W,), jnp.int32),
            pltpu.VMEM((CHUNK,), jnp.int32),
            pltpu.VMEM((CHUNK,), jnp.int32),
            pltpu.VMEM((CHUNK,), jnp.int32),
            pltpu.VMEM((CHUNK, 128), jnp.float32),
            pltpu.VMEM((CHUNK, 128), jnp.float32),
            pltpu.VMEM((CHUNK, 128), jnp.float32),
            pltpu.VMEM_SHARED((N_NODES, 128), jnp.float32),
        ] + [pltpu.SemaphoreType.DMA] * 9,
        mesh=plsc.VectorSubcoreMesh(**_MESH),
        compiler_params=_SC_PARAMS,
    )
    return k(g, src32, dst32)


def _tc_dinv_body(h_ref, o_ref):
    deg = jnp.sum(h_ref[...], axis=0).astype(jnp.float32) + 1.0
    o_ref[...] = lax.rsqrt(deg)


def _tc_dinv(hists):
    return pl.pallas_call(
        _tc_dinv_body,
        out_shape=jax.ShapeDtypeStruct((N_NODES,), jnp.float32),
    )(hists)


_BLK = 1000


def _tc_scale_body(d_ref, x_ref, o_ref):
    o_ref[...] = d_ref[...] * x_ref[...]


def _tc_scale(dinv_col, x):
    grid = (N_NODES // _BLK,)
    return pl.pallas_call(
        _tc_scale_body,
        grid=grid,
        in_specs=[
            pl.BlockSpec((_BLK, 1), lambda i: (i, 0)),
            pl.BlockSpec((_BLK, 128), lambda i: (i, 0)),
        ],
        out_specs=pl.BlockSpec((_BLK, 128), lambda i: (i, 0)),
        out_shape=jax.ShapeDtypeStruct((N_NODES, 128), jnp.float32),
    )(dinv_col, x)


def _tc_mid_body(p0, p1, g1, d, w1, bb1, w2, o_ref):
    a = d[...] * (p0[...] + p1[...] + g1[...])
    h = jnp.dot(a, w1[...], preferred_element_type=jnp.float32) + bb1[...]
    h = jnp.maximum(h, 0.0)
    t = jnp.dot(h, w2[...], preferred_element_type=jnp.float32)
    o_ref[...] = d[...] * t


def _tc_mid(p0, p1, g1, dinv_col, W1, b1, W2):
    grid = (N_NODES // _BLK,)
    row = lambda i: (i, 0)
    full = lambda i: (0, 0)
    return pl.pallas_call(
        _tc_mid_body,
        grid=grid,
        in_specs=[
            pl.BlockSpec((_BLK, 128), row),
            pl.BlockSpec((_BLK, 128), row),
            pl.BlockSpec((_BLK, 128), row),
            pl.BlockSpec((_BLK, 1), row),
            pl.BlockSpec((128, 256), full),
            pl.BlockSpec((1, 256), full),
            pl.BlockSpec((256, 128), full),
        ],
        out_specs=pl.BlockSpec((_BLK, 128), row),
        out_shape=jax.ShapeDtypeStruct((N_NODES, 128), jnp.float32),
    )(p0, p1, g1, dinv_col, W1, b1.reshape(1, 256), W2)


def _tc_post_body(q0, q1, g2, d, bb2, o_ref):
    a = d[...] * (q0[...] + q1[...] + g2[...]) + bb2[...]
    m = jnp.max(a, axis=-1, keepdims=True)
    e = jnp.exp(a - m)
    o_ref[...] = e / jnp.sum(e, axis=-1, keepdims=True)


def _tc_post(q0, q1, g2, dinv_col, b2):
    grid = (N_NODES // _BLK,)
    row = lambda i: (i, 0)
    full = lambda i: (0, 0)
    return pl.pallas_call(
        _tc_post_body,
        grid=grid,
        in_specs=[
            pl.BlockSpec((_BLK, 128), row),
            pl.BlockSpec((_BLK, 128), row),
            pl.BlockSpec((_BLK, 128), row),
            pl.BlockSpec((_BLK, 1), row),
            pl.BlockSpec((1, 128), full),
        ],
        out_specs=pl.BlockSpec((_BLK, 128), row),
        out_shape=jax.ShapeDtypeStruct((N_NODES, 128), jnp.float32),
    )(q0, q1, g2, dinv_col, b2.reshape(1, 128))


def kernel(x, edge_index, W1, b1, W2, b2):
    src = edge_index[0].astype(jnp.int32)
    dst = edge_index[1].astype(jnp.int32)
    src_r = src.reshape(N_WORKERS, E_PER_W)
    dst_r = dst.reshape(N_WORKERS * N_CHUNKS, CHUNK)
    dst_flat = dst.reshape(N_WORKERS, E_PER_W)

    hists = _sc_deg(dst_flat)
    dinv = _tc_dinv(hists)
    dinv_col = dinv.reshape(N_NODES, 1)

    g1 = _tc_scale(dinv_col, x)
    p = _sc_agg(g1, src_r, dst_r).reshape(2, N_NODES, 128)
    g2 = _tc_mid(p[0], p[1], g1, dinv_col, W1, b1, W2)
    q = _sc_agg(g2, src_r, dst_r).reshape(2, N_NODES, 128)
    out = _tc_post(q[0], q[1], g2, dinv_col, b2)
    return out


# speedup vs baseline: 1.2475x; 1.0316x over previous
import jax
import jax.numpy as jnp
from jax import lax
from jax.experimental import pallas as pl
from jax.experimental.pallas import tpu as pltpu
from jax.experimental.pallas import tpu_sc as plsc

N_NODES = 10000
N_EDGES = 320000
N_WORKERS = 32
E_PER_W = N_EDGES // N_WORKERS
CHUNK = 80
N_CHUNKS = E_PER_W // CHUNK
ROWS_PER_TILE = N_NODES // 16

_MESH = dict(core_axis_name="c", subcore_axis_name="s")
_SC_PARAMS = pltpu.CompilerParams(needs_layout_passes=False)


def _sc_deg_body(dst_hbm, out_hbm, dstv, hist):
    c = lax.axis_index("c")
    s = lax.axis_index("s")
    wid = s * 2 + c
    pltpu.sync_copy(dst_hbm.at[wid], dstv)

    def zero(i, _):
        hist[pl.ds(i * 16, 16)] = jnp.zeros((16,), jnp.int32)
        return 0

    lax.fori_loop(0, N_NODES // 16, zero, 0)

    ones = jnp.ones((16,), jnp.int32)

    def step(i, _):
        idx = dstv[pl.ds(i * 16, 16)]
        plsc.addupdate_scatter(hist, [idx], ones)
        return 0

    lax.fori_loop(0, E_PER_W // 16, step, 0)
    pltpu.sync_copy(hist, out_hbm.at[wid])


def _sc_deg(dst32):
    k = pl.kernel(
        _sc_deg_body,
        out_type=jax.ShapeDtypeStruct((N_WORKERS, N_NODES), jnp.int32),
        scratch_types=[
            pltpu.VMEM((E_PER_W,), jnp.int32),
            pltpu.VMEM((N_NODES,), jnp.int32),
        ],
        mesh=plsc.VectorSubcoreMesh(**_MESH),
        compiler_params=_SC_PARAMS,
    )
    return k(dst32)


def _sc_agg_body(g_hbm, src_hbm, dst_hbm, out_hbm, srcv,
                 d0, d1, d2, b0, b1, b2, acc,
                 sg0, sg1, sg2, sd0, sd1, sd2, ss0, ss1, ss2):
    c = lax.axis_index("c")
    s = lax.axis_index("s")
    wid = s * 2 + c
    pltpu.sync_copy(src_hbm.at[wid], srcv)

    def zbuf(r, _):
        for k in range(8):
            b0[r, pl.ds(k * 16, 16)] = jnp.zeros((16,), jnp.float32)
        return 0

    lax.fori_loop(0, CHUNK, zbuf, 0)
    base = s * ROWS_PER_TILE
    for t in range(7):
        pltpu.sync_copy(b0, acc.at[pl.ds(base + t * CHUNK, CHUNK)])
    pltpu.sync_copy(b0.at[pl.ds(0, 65)], acc.at[pl.ds(base + 560, 65)])
    plsc.subcore_barrier()

    D = (d0, d1, d2)
    B = (b0, b1, b2)
    SG = (sg0, sg1, sg2)
    SD = (sd0, sd1, sd2)
    SS = (ss0, ss1, ss2)

    def g_at(j):
        return g_hbm.at[srcv.at[pl.ds(j * CHUNK, CHUNK)]]

    def dst_at(j):
        return dst_hbm.at[wid * N_CHUNKS + j]

    def issue_gd(m, k):
        pltpu.async_copy(dst_at(m), D[k], SD[k])
        pltpu.async_copy(g_at(m), B[k], SG[k])

    def wait_gd(m, k):
        pltpu.make_async_copy(g_at(m), B[k], SG[k]).wait()
        pltpu.make_async_copy(dst_at(m), D[k], SD[k]).wait()

    def issue_s(k):
        pltpu.async_copy(B[k], acc.at[D[k]], SS[k], add=True)

    def wait_s(k):
        pltpu.make_async_copy(B[k], acc.at[D[k]], SS[k]).wait()

    issue_gd(0, 0)
    issue_gd(1, 1)
    wait_gd(0, 0)
    issue_s(0)
    issue_gd(2, 2)
    wait_gd(1, 1)
    issue_s(1)
    wait_s(0)
    issue_gd(3, 0)
    wait_gd(2, 2)
    issue_s(2)
    wait_s(1)
    issue_gd(4, 1)
    wait_gd(3, 0)
    issue_s(0)

    def tri(i, _):
        m = 4 + 3 * i
        wait_s(2)
        issue_gd(m + 1, 2)
        wait_gd(m, 1)
        issue_s(1)
        wait_s(0)
        issue_gd(m + 2, 0)
        wait_gd(m + 1, 2)
        issue_s(2)
        wait_s(1)
        issue_gd(m + 3, 1)
        wait_gd(m + 2, 0)
        issue_s(0)
        return 0

    lax.fori_loop(0, 40, tri, 0)
    wait_s(2)
    wait_gd(124, 1)
    issue_s(1)
    wait_s(0)
    wait_s(1)
    plsc.subcore_barrier()
    pltpu.sync_copy(acc.at[pl.ds(base, ROWS_PER_TILE)], out_hbm.at[c, s])


def _sc_agg(g, src32, dst32):
    k = pl.kernel(
        _sc_agg_body,
        out_type=jax.ShapeDtypeStruct((2, 16, ROWS_PER_TILE, 128), jnp.float32),
        scratch_types=[
            pltpu.VMEM((E_PER_W,), jnp.int32),
            pltpu.VMEM((CHUNK,), jnp.int32),
            pltpu.VMEM((CHUNK,), jnp.int32),
            pltpu.VMEM((CHUNK,), jnp.int32),
            pltpu.VMEM((CHUNK, 128), jnp.float32),
            pltpu.VMEM((CHUNK, 128), jnp.float32),
            pltpu.VMEM((CHUNK, 128), jnp.float32),
            pltpu.VMEM_SHARED((N_NODES, 128), jnp.float32),
        ] + [pltpu.SemaphoreType.DMA] * 9,
        mesh=plsc.VectorSubcoreMesh(**_MESH),
        compiler_params=_SC_PARAMS,
    )
    return k(g, src32, dst32)


_BLK = 1000


def _tc_pre_body(h_ref, x_ref, d_ref, g_ref):
    deg = jnp.sum(h_ref[...], axis=0).astype(jnp.float32) + 1.0
    dinv = lax.rsqrt(deg).reshape(N_NODES, 1)
    d_ref[...] = dinv
    g_ref[...] = dinv * x_ref[...]


def _tc_pre(hists, x):
    return pl.pallas_call(
        _tc_pre_body,
        out_shape=[
            jax.ShapeDtypeStruct((N_NODES, 1), jnp.float32),
            jax.ShapeDtypeStruct((N_NODES, 128), jnp.float32),
        ],
    )(hists, x)


def _tc_mid_body(p0, p1, g1, d, w1, bb1, w2, o_ref):
    a = d[...] * (p0[...] + p1[...] + g1[...])
    h = jnp.dot(a, w1[...], preferred_element_type=jnp.float32) + bb1[...]
    h = jnp.maximum(h, 0.0)
    t = jnp.dot(h, w2[...], preferred_element_type=jnp.float32)
    o_ref[...] = d[...] * t


def _tc_mid(p0, p1, g1, dinv_col, W1, b1, W2):
    grid = (N_NODES // _BLK,)
    row = lambda i: (i, 0)
    full = lambda i: (0, 0)
    return pl.pallas_call(
        _tc_mid_body,
        grid=grid,
        in_specs=[
            pl.BlockSpec((_BLK, 128), row),
            pl.BlockSpec((_BLK, 128), row),
            pl.BlockSpec((_BLK, 128), row),
            pl.BlockSpec((_BLK, 1), row),
            pl.BlockSpec((128, 256), full),
            pl.BlockSpec((1, 256), full),
            pl.BlockSpec((256, 128), full),
        ],
        out_specs=pl.BlockSpec((_BLK, 128), row),
        out_shape=jax.ShapeDtypeStruct((N_NODES, 128), jnp.float32),
    )(p0, p1, g1, dinv_col, W1, b1.reshape(1, 256), W2)


def _tc_post_body(q0, q1, g2, d, bb2, o_ref):
    a = d[...] * (q0[...] + q1[...] + g2[...]) + bb2[...]
    m = jnp.max(a, axis=-1, keepdims=True)
    e = jnp.exp(a - m)
    o_ref[...] = e / jnp.sum(e, axis=-1, keepdims=True)


def _tc_post(q0, q1, g2, dinv_col, b2):
    grid = (N_NODES // _BLK,)
    row = lambda i: (i, 0)
    full = lambda i: (0, 0)
    return pl.pallas_call(
        _tc_post_body,
        grid=grid,
        in_specs=[
            pl.BlockSpec((_BLK, 128), row),
            pl.BlockSpec((_BLK, 128), row),
            pl.BlockSpec((_BLK, 128), row),
            pl.BlockSpec((_BLK, 1), row),
            pl.BlockSpec((1, 128), full),
        ],
        out_specs=pl.BlockSpec((_BLK, 128), row),
        out_shape=jax.ShapeDtypeStruct((N_NODES, 128), jnp.float32),
    )(q0, q1, g2, dinv_col, b2.reshape(1, 128))


def kernel(x, edge_index, W1, b1, W2, b2):
    src = edge_index[0].astype(jnp.int32)
    dst = edge_index[1].astype(jnp.int32)
    src_r = src.reshape(N_WORKERS, E_PER_W)
    dst_r = dst.reshape(N_WORKERS * N_CHUNKS, CHUNK)
    dst_flat = dst.reshape(N_WORKERS, E_PER_W)

    hists = _sc_deg(dst_flat)
    dinv_col, g1 = _tc_pre(hists, x)
    p = _sc_agg(g1, src_r, dst_r).reshape(2, N_NODES, 128)
    g2 = _tc_mid(p[0], p[1], g1, dinv_col, W1, b1, W2)
    q = _sc_agg(g2, src_r, dst_r).reshape(2, N_NODES, 128)
    out = _tc_post(q[0], q[1], g2, dinv_col, b2)
    return out
